# Initial kernel scaffold; baseline (speedup 1.0000x reference)
import jax
import jax.numpy as jnp
from jax.experimental import pallas as pl


def kernel(grid_node_features, mesh_node_features, grid2mesh_edge_features, edge_index, halo_indices, num_local, eW1, eb1, eW2, eb2, eg, ebn, nW1, nb1, nW2, nb2, ng, nbn, gW1, gb1, gW2, gb2, gg, gbn):
    raise NotImplementedError("write your pallas kernel here")


# speedup vs baseline: 2.6661x; 2.6661x over previous
import functools

import jax
import jax.numpy as jnp
from jax import lax
from jax.experimental import pallas as pl
from jax.experimental.pallas import tpu as pltpu
from jax.experimental.pallas import tpu_sc as plsc

D = 128
CHUNK = 128
NC, NS = 2, 16
NW = NC * NS
ACC_PAD = 10240
LN_EPS = 1e-5


def _mlp2_body(x_ref, a_ref, w1x_ref, w1a_ref, b1_ref, w2_ref, b2_ref,
               g_ref, bsh_ref, out_ref):
    x = x_ref[...]
    pre = jnp.dot(x, w1x_ref[...], preferred_element_type=jnp.float32) + b1_ref[...]
    if a_ref is not None:
        pre = pre + a_ref
    h = pre * (1.0 / (1.0 + jnp.exp(-pre)))
    y = jnp.dot(h, w2_ref[...], preferred_element_type=jnp.float32) + b2_ref[...]
    mu = jnp.mean(y, axis=-1, keepdims=True)
    yc = y - mu
    var = jnp.mean(yc * yc, axis=-1, keepdims=True)
    out_ref[...] = x + yc * lax.rsqrt(var + LN_EPS) * g_ref[...] + bsh_ref[...]


def _resid_mlp(x, w1, b1, w2, b2, g, bsh, blk):
    n = x.shape[0]
    body = lambda x_ref, w1x, b1_, w2_, b2_, g_, bsh_, out_ref: _mlp2_body(
        x_ref, None, w1x, None, b1_, w2_, b2_, g_, bsh_, out_ref)
    wspec = pl.BlockSpec((D, D), lambda i: (0, 0))
    vspec = pl.BlockSpec((1, D), lambda i: (0, 0))
    return pl.pallas_call(
        body,
        grid=(n // blk,),
        in_specs=[pl.BlockSpec((blk, D), lambda i: (i, 0)),
                  wspec, vspec, wspec, vspec, vspec, vspec],
        out_specs=pl.BlockSpec((blk, D), lambda i: (i, 0)),
        out_shape=jax.ShapeDtypeStruct((n, D), jnp.float32),
    )(x, w1, b1.reshape(1, D), w2, b2.reshape(1, D), g.reshape(1, D),
      bsh.reshape(1, D))


def _prep_body(m_ref, ws_ref, wd_ref, b1_ref, ts_ref, td_ref):
    m = m_ref[...]
    ts_ref[...] = jnp.dot(m, ws_ref[...], preferred_element_type=jnp.float32) + b1_ref[...]
    td_ref[...] = jnp.dot(m, wd_ref[...], preferred_element_type=jnp.float32)


def _prep(mesh, ws, wd, b1):
    n = mesh.shape[0]
    blk = 2000
    wspec = pl.BlockSpec((D, D), lambda i: (0, 0))
    return pl.pallas_call(
        _prep_body,
        grid=(n // blk,),
        in_specs=[pl.BlockSpec((blk, D), lambda i: (i, 0)), wspec, wspec,
                  pl.BlockSpec((1, D), lambda i: (0, 0))],
        out_specs=[pl.BlockSpec((blk, D), lambda i: (i, 0))] * 2,
        out_shape=[jax.ShapeDtypeStruct((n, D), jnp.float32)] * 2,
    )(mesh, ws, wd, b1.reshape(1, D))


def _edge_body(g_ref, e_ref, we_ref, w2_ref, b2_ref, gam_ref, bet_ref, out_ref):
    e = e_ref[...]
    pre = g_ref[...] + jnp.dot(e, we_ref[...], preferred_element_type=jnp.float32)
    h = pre * (1.0 / (1.0 + jnp.exp(-pre)))
    y = jnp.dot(h, w2_ref[...], preferred_element_type=jnp.float32) + b2_ref[...]
    mu = jnp.mean(y, axis=-1, keepdims=True)
    yc = y - mu
    var = jnp.mean(yc * yc, axis=-1, keepdims=True)
    out_ref[...] = e + yc * lax.rsqrt(var + LN_EPS) * gam_ref[...] + bet_ref[...]


def _edge_mlp(g, e, we, w2, b2, gam, bet):
    n = e.shape[0]
    blk = 2000
    wspec = pl.BlockSpec((D, D), lambda i: (0, 0))
    vspec = pl.BlockSpec((1, D), lambda i: (0, 0))
    return pl.pallas_call(
        _edge_body,
        grid=(n // blk,),
        in_specs=[pl.BlockSpec((blk, D), lambda i: (i, 0)),
                  pl.BlockSpec((blk, D), lambda i: (i, 0)),
                  wspec, wspec, vspec, vspec, vspec],
        out_specs=pl.BlockSpec((blk, D), lambda i: (i, 0)),
        out_shape=jax.ShapeDtypeStruct((n, D), jnp.float32),
    )(g, e, we, w2, b2.reshape(1, D), gam.reshape(1, D), bet.reshape(1, D))


def _node_body(x_ref, p0_ref, p1_ref, w1x_ref, w1a_ref, b1_ref, w2_ref,
               b2_ref, g_ref, bsh_ref, out_ref):
    agg = p0_ref[0] + p1_ref[0]
    apre = jnp.dot(agg, w1a_ref[...], preferred_element_type=jnp.float32)
    _mlp2_body(x_ref, apre, w1x_ref, None, b1_ref, w2_ref, b2_ref, g_ref,
               bsh_ref, out_ref)


def _node_mlp(mesh, partials, w1x, w1a, b1, w2, b2, g, bsh):
    n = mesh.shape[0]
    blk = 2000
    wspec = pl.BlockSpec((D, D), lambda i: (0, 0))
    vspec = pl.BlockSpec((1, D), lambda i: (0, 0))
    return pl.pallas_call(
        _node_body,
        grid=(n // blk,),
        in_specs=[pl.BlockSpec((blk, D), lambda i: (i, 0)),
                  pl.BlockSpec((1, blk, D), lambda i: (0, i, 0)),
                  pl.BlockSpec((1, blk, D), lambda i: (1, i, 0)),
                  wspec, wspec, vspec, wspec, vspec, vspec, vspec],
        out_specs=pl.BlockSpec((blk, D), lambda i: (i, 0)),
        out_shape=jax.ShapeDtypeStruct((n, D), jnp.float32),
    )(mesh, partials, partials, w1x, w1a, b1.reshape(1, D), w2,
      b2.reshape(1, D), g.reshape(1, D), bsh.reshape(1, D))


def _sc_gather(src_idx, dst_idx, aug_map, tsm, tdm):
    n_edge = src_idx.shape[0]
    n_aug = aug_map.shape[0]
    nchunks = n_edge // CHUNK
    iters = (nchunks + NW - 1) // NW
    mesh_sc = plsc.VectorSubcoreMesh(core_axis_name="c", subcore_axis_name="s")

    def body(src_hbm, dst_hbm, map_hbm, ts_hbm, td_hbm, out_hbm,
             map_v, si_v, di_v, ms_v, md_v, rs_v, rd_v, sem1, sem2):
        cid = lax.axis_index("c")
        sid = lax.axis_index("s")
        wid = sid * NC + cid
        pltpu.sync_copy(map_hbm, map_v)

        def chunk_body(it, carry):
            ci = wid + it * NW

            @pl.when(ci < nchunks)
            def _():
                base = ci * CHUNK
                pltpu.sync_copy(src_hbm.at[pl.ds(base, CHUNK)], si_v)
                pltpu.sync_copy(dst_hbm.at[pl.ds(base, CHUNK)], di_v)
                for j in range(CHUNK // 16):
                    sl = pl.ds(j * 16, 16)
                    ms_v[sl] = plsc.load_gather(map_v, [si_v[sl]])
                    md_v[sl] = plsc.load_gather(map_v, [di_v[sl]])
                c1 = pltpu.async_copy(ts_hbm.at[ms_v], rs_v, sem1)
                c2 = pltpu.async_copy(td_hbm.at[md_v], rd_v, sem2)
                c1.wait()
                c2.wait()

                def add_body(i, c):
                    r = i // 8
                    k = (i % 8) * 16
                    rs_v[r, pl.ds(k, 16)] = (rs_v[r, pl.ds(k, 16)]
                                             + rd_v[r, pl.ds(k, 16)])
                    return c

                lax.fori_loop(0, CHUNK * 8, add_body, 0)
                pltpu.sync_copy(rs_v, out_hbm.at[pl.ds(base, CHUNK)])

            return carry

        lax.fori_loop(0, iters, chunk_body, 0)

    call = pl.kernel(
        body,
        out_type=jax.ShapeDtypeStruct((n_edge, D), jnp.float32),
        mesh=mesh_sc,
        compiler_params=pltpu.CompilerParams(needs_layout_passes=False),
        scratch_types=[
            pltpu.VMEM((n_aug,), jnp.int32),
            pltpu.VMEM((CHUNK,), jnp.int32),
            pltpu.VMEM((CHUNK,), jnp.int32),
            pltpu.VMEM((CHUNK,), jnp.int32),
            pltpu.VMEM((CHUNK,), jnp.int32),
            pltpu.VMEM((CHUNK, D), jnp.float32),
            pltpu.VMEM((CHUNK, D), jnp.float32),
            pltpu.SemaphoreType.DMA,
            pltpu.SemaphoreType.DMA,
        ],
    )
    return call(src_idx, dst_idx, aug_map, tsm, tdm)


def _sc_scatter(ef, dst_idx, n_mesh):
    n_edge = ef.shape[0]
    nchunks = n_edge // CHUNK
    iters = (nchunks + NW - 1) // NW
    zrows = ACC_PAD // NS
    mesh_sc = plsc.VectorSubcoreMesh(core_axis_name="c", subcore_axis_name="s")

    def body(ef_hbm, di_hbm, out_hbm, acc_sh, ids_v, mp_v, rows_v, zb_v):
        cid = lax.axis_index("c")
        sid = lax.axis_index("s")
        wid = sid * NC + cid

        def zb(i, c):
            r = i // 8
            k = (i % 8) * 16
            zb_v[r, pl.ds(k, 16)] = jnp.zeros((16,), jnp.float32)
            return c

        lax.fori_loop(0, 64 * 8, zb, 0)

        def zc(i, c):
            pltpu.sync_copy(zb_v, acc_sh.at[pl.ds(sid * zrows + i * 64, 64)])
            return c

        lax.fori_loop(0, zrows // 64, zc, 0)
        plsc.subcore_barrier()

        def chunk_body(it, carry):
            ci = wid + it * NW

            @pl.when(ci < nchunks)
            def _():
                base = ci * CHUNK
                pltpu.sync_copy(di_hbm.at[pl.ds(base, CHUNK)], ids_v)
                pltpu.sync_copy(ef_hbm.at[pl.ds(base, CHUNK)], rows_v)
                for j in range(CHUNK // 16):
                    sl = pl.ds(j * 16, 16)
                    v = ids_v[sl]
                    mp_v[sl] = jnp.where(v < n_mesh, v, n_mesh)
                pltpu.sync_copy(rows_v, acc_sh.at[mp_v], add=True)

            return carry

        lax.fori_loop(0, iters, chunk_body, 0)
        plsc.subcore_barrier()
        pltpu.sync_copy(acc_sh.at[pl.ds(sid * zrows, zrows)],
                        out_hbm.at[cid, pl.ds(sid * zrows, zrows)])

    call = pl.kernel(
        body,
        out_type=jax.ShapeDtypeStruct((NC, ACC_PAD, D), jnp.float32),
        mesh=mesh_sc,
        scratch_types=[
            pltpu.VMEM_SHARED((ACC_PAD, D), jnp.float32),
            pltpu.VMEM((CHUNK,), jnp.int32),
            pltpu.VMEM((CHUNK,), jnp.int32),
            pltpu.VMEM((CHUNK, D), jnp.float32),
            pltpu.VMEM((64, D), jnp.float32),
        ],
    )
    return call(ef, dst_idx)


def kernel(grid_node_features, mesh_node_features, grid2mesh_edge_features,
           edge_index, halo_indices, num_local,
           eW1, eb1, eW2, eb2, eg, ebn,
           nW1, nb1, nW2, nb2, ng, nbn,
           gW1, gb1, gW2, gb2, gg, gbn):
    n_mesh = mesh_node_features.shape[0]
    dst_idx = edge_index[:, 0]
    src_idx = edge_index[:, 1]
    aug_map = jnp.concatenate([jnp.arange(n_mesh, dtype=jnp.int32),
                               halo_indices.astype(jnp.int32)])
    ws, wd, we = eW1[:D], eW1[D:2 * D], eW1[2 * D:]

    tsm, tdm = _prep(mesh_node_features, ws, wd, eb1)
    g = _sc_gather(src_idx, dst_idx, aug_map, tsm, tdm)
    ef = _edge_mlp(g, grid2mesh_edge_features, we, eW2, eb2, eg, ebn)
    partials = _sc_scatter(ef, dst_idx, n_mesh)
    mesh_out = _node_mlp(mesh_node_features, partials, nW1[:D], nW1[D:],
                         nb1, nW2, nb2, ng, nbn)
    grid_out = _resid_mlp(grid_node_features, gW1, gb1, gW2, gb2, gg, gbn,
                          blk=2000)
    return (grid_out, mesh_out)


# speedup vs baseline: 3.4695x; 1.3013x over previous
import functools

import jax
import jax.numpy as jnp
from jax import lax
from jax.experimental import pallas as pl
from jax.experimental.pallas import tpu as pltpu
from jax.experimental.pallas import tpu_sc as plsc

D = 128
CHUNK = 128
NC, NS = 2, 16
NW = NC * NS
ACC_PAD = 10240
LN_EPS = 1e-5


def _mlp2_body(x_ref, a_ref, w1x_ref, w1a_ref, b1_ref, w2_ref, b2_ref,
               g_ref, bsh_ref, out_ref):
    x = x_ref[...]
    pre = jnp.dot(x, w1x_ref[...], preferred_element_type=jnp.float32) + b1_ref[...]
    if a_ref is not None:
        pre = pre + a_ref
    h = pre * (1.0 / (1.0 + jnp.exp(-pre)))
    y = jnp.dot(h, w2_ref[...], preferred_element_type=jnp.float32) + b2_ref[...]
    mu = jnp.mean(y, axis=-1, keepdims=True)
    yc = y - mu
    var = jnp.mean(yc * yc, axis=-1, keepdims=True)
    out_ref[...] = x + yc * lax.rsqrt(var + LN_EPS) * g_ref[...] + bsh_ref[...]


def _resid_mlp(x, w1, b1, w2, b2, g, bsh, blk):
    n = x.shape[0]
    body = lambda x_ref, w1x, b1_, w2_, b2_, g_, bsh_, out_ref: _mlp2_body(
        x_ref, None, w1x, None, b1_, w2_, b2_, g_, bsh_, out_ref)
    wspec = pl.BlockSpec((D, D), lambda i: (0, 0))
    vspec = pl.BlockSpec((1, D), lambda i: (0, 0))
    return pl.pallas_call(
        body,
        grid=(n // blk,),
        in_specs=[pl.BlockSpec((blk, D), lambda i: (i, 0)),
                  wspec, vspec, wspec, vspec, vspec, vspec],
        out_specs=pl.BlockSpec((blk, D), lambda i: (i, 0)),
        out_shape=jax.ShapeDtypeStruct((n, D), jnp.float32),
    )(x, w1, b1.reshape(1, D), w2, b2.reshape(1, D), g.reshape(1, D),
      bsh.reshape(1, D))


def _prep_body(m_ref, ws_ref, wd_ref, b1_ref, ts_ref, td_ref):
    m = m_ref[...]
    ts_ref[...] = jnp.dot(m, ws_ref[...], preferred_element_type=jnp.float32) + b1_ref[...]
    td_ref[...] = jnp.dot(m, wd_ref[...], preferred_element_type=jnp.float32)


def _prep(mesh, ws, wd, b1):
    n = mesh.shape[0]
    blk = 2000
    wspec = pl.BlockSpec((D, D), lambda i: (0, 0))
    return pl.pallas_call(
        _prep_body,
        grid=(n // blk,),
        in_specs=[pl.BlockSpec((blk, D), lambda i: (i, 0)), wspec, wspec,
                  pl.BlockSpec((1, D), lambda i: (0, 0))],
        out_specs=[pl.BlockSpec((blk, D), lambda i: (i, 0))] * 2,
        out_shape=[jax.ShapeDtypeStruct((n, D), jnp.float32)] * 2,
    )(mesh, ws, wd, b1.reshape(1, D))


def _edge_body(g_ref, e_ref, we_ref, w2_ref, b2_ref, gam_ref, bet_ref, out_ref):
    e = e_ref[...]
    pre = g_ref[...] + jnp.dot(e, we_ref[...], preferred_element_type=jnp.float32)
    h = pre * (1.0 / (1.0 + jnp.exp(-pre)))
    y = jnp.dot(h, w2_ref[...], preferred_element_type=jnp.float32) + b2_ref[...]
    mu = jnp.mean(y, axis=-1, keepdims=True)
    yc = y - mu
    var = jnp.mean(yc * yc, axis=-1, keepdims=True)
    out_ref[...] = e + yc * lax.rsqrt(var + LN_EPS) * gam_ref[...] + bet_ref[...]


def _edge_mlp(g, e, we, w2, b2, gam, bet):
    n = e.shape[0]
    blk = 2000
    wspec = pl.BlockSpec((D, D), lambda i: (0, 0))
    vspec = pl.BlockSpec((1, D), lambda i: (0, 0))
    return pl.pallas_call(
        _edge_body,
        grid=(n // blk,),
        in_specs=[pl.BlockSpec((blk, D), lambda i: (i, 0)),
                  pl.BlockSpec((blk, D), lambda i: (i, 0)),
                  wspec, wspec, vspec, vspec, vspec],
        out_specs=pl.BlockSpec((blk, D), lambda i: (i, 0)),
        out_shape=jax.ShapeDtypeStruct((n, D), jnp.float32),
    )(g, e, we, w2, b2.reshape(1, D), gam.reshape(1, D), bet.reshape(1, D))


def _node_body(x_ref, p0_ref, p1_ref, w1x_ref, w1a_ref, b1_ref, w2_ref,
               b2_ref, g_ref, bsh_ref, out_ref):
    agg = p0_ref[0] + p1_ref[0]
    apre = jnp.dot(agg, w1a_ref[...], preferred_element_type=jnp.float32)
    _mlp2_body(x_ref, apre, w1x_ref, None, b1_ref, w2_ref, b2_ref, g_ref,
               bsh_ref, out_ref)


def _node_mlp(mesh, partials, w1x, w1a, b1, w2, b2, g, bsh):
    n = mesh.shape[0]
    blk = 2000
    wspec = pl.BlockSpec((D, D), lambda i: (0, 0))
    vspec = pl.BlockSpec((1, D), lambda i: (0, 0))
    return pl.pallas_call(
        _node_body,
        grid=(n // blk,),
        in_specs=[pl.BlockSpec((blk, D), lambda i: (i, 0)),
                  pl.BlockSpec((1, blk, D), lambda i: (0, i, 0)),
                  pl.BlockSpec((1, blk, D), lambda i: (1, i, 0)),
                  wspec, wspec, vspec, wspec, vspec, vspec, vspec],
        out_specs=pl.BlockSpec((blk, D), lambda i: (i, 0)),
        out_shape=jax.ShapeDtypeStruct((n, D), jnp.float32),
    )(mesh, partials, partials, w1x, w1a, b1.reshape(1, D), w2,
      b2.reshape(1, D), g.reshape(1, D), bsh.reshape(1, D))


def _sc_gather(src_idx, dst_idx, aug_map, tsm, tdm):
    n_edge = src_idx.shape[0]
    n_aug = aug_map.shape[0]
    nchunks = n_edge // CHUNK
    iters = (nchunks + NW - 1) // NW
    mesh_sc = plsc.VectorSubcoreMesh(core_axis_name="c", subcore_axis_name="s")

    def body(src_hbm, dst_hbm, map_hbm, ts_hbm, td_hbm, out_hbm,
             map_v, si_v, di_v, ms_v, md_v, rs_v, rd_v, sem1, sem2):
        cid = lax.axis_index("c")
        sid = lax.axis_index("s")
        wid = sid * NC + cid
        pltpu.sync_copy(map_hbm, map_v)

        def chunk_body(it, carry):
            ci = wid + it * NW

            @pl.when(ci < nchunks)
            def _():
                base = ci * CHUNK
                pltpu.sync_copy(src_hbm.at[pl.ds(base, CHUNK)], si_v)
                pltpu.sync_copy(dst_hbm.at[pl.ds(base, CHUNK)], di_v)
                for j in range(CHUNK // 16):
                    sl = pl.ds(j * 16, 16)
                    ms_v[sl] = plsc.load_gather(map_v, [si_v[sl]])
                    md_v[sl] = plsc.load_gather(map_v, [di_v[sl]])
                c1 = pltpu.async_copy(ts_hbm.at[ms_v], rs_v, sem1)
                c2 = pltpu.async_copy(td_hbm.at[md_v], rd_v, sem2)
                c1.wait()
                c2.wait()

                @plsc.parallel_loop(0, CHUNK, 1, unroll=4)
                def _add(r):
                    for k in range(8):
                        sl = pl.ds(k * 16, 16)
                        plsc.addupdate(rs_v.at[r, sl], rd_v[r, sl])
                pltpu.sync_copy(rs_v, out_hbm.at[pl.ds(base, CHUNK)])

            return carry

        lax.fori_loop(0, iters, chunk_body, 0)

    call = pl.kernel(
        body,
        out_type=jax.ShapeDtypeStruct((n_edge, D), jnp.float32),
        mesh=mesh_sc,
        compiler_params=pltpu.CompilerParams(needs_layout_passes=False),
        scratch_types=[
            pltpu.VMEM((n_aug,), jnp.int32),
            pltpu.VMEM((CHUNK,), jnp.int32),
            pltpu.VMEM((CHUNK,), jnp.int32),
            pltpu.VMEM((CHUNK,), jnp.int32),
            pltpu.VMEM((CHUNK,), jnp.int32),
            pltpu.VMEM((CHUNK, D), jnp.float32),
            pltpu.VMEM((CHUNK, D), jnp.float32),
            pltpu.SemaphoreType.DMA,
            pltpu.SemaphoreType.DMA,
        ],
    )
    return call(src_idx, dst_idx, aug_map, tsm, tdm)


def _sc_scatter(ef, dst_idx, n_mesh):
    n_edge = ef.shape[0]
    nchunks = n_edge // CHUNK
    iters = (nchunks + NW - 1) // NW
    zrows = ACC_PAD // NS
    mesh_sc = plsc.VectorSubcoreMesh(core_axis_name="c", subcore_axis_name="s")

    def body(ef_hbm, di_hbm, out_hbm, acc_sh, ids_v, mp_v, rows_v, zb_v):
        cid = lax.axis_index("c")
        sid = lax.axis_index("s")
        wid = sid * NC + cid

        @plsc.parallel_loop(0, 64, 1, unroll=4)
        def _zb(r):
            for k in range(8):
                zb_v[r, pl.ds(k * 16, 16)] = jnp.zeros((16,), jnp.float32)

        def zc(i, c):
            pltpu.sync_copy(zb_v, acc_sh.at[pl.ds(sid * zrows + i * 64, 64)])
            return c

        lax.fori_loop(0, zrows // 64, zc, 0)
        plsc.subcore_barrier()

        def chunk_body(it, carry):
            ci = wid + it * NW

            @pl.when(ci < nchunks)
            def _():
                base = ci * CHUNK
                pltpu.sync_copy(di_hbm.at[pl.ds(base, CHUNK)], ids_v)
                pltpu.sync_copy(ef_hbm.at[pl.ds(base, CHUNK)], rows_v)
                for j in range(CHUNK // 16):
                    sl = pl.ds(j * 16, 16)
                    v = ids_v[sl]
                    mp_v[sl] = jnp.where(v < n_mesh, v, n_mesh)
                pltpu.sync_copy(rows_v, acc_sh.at[mp_v], add=True)

            return carry

        lax.fori_loop(0, iters, chunk_body, 0)
        plsc.subcore_barrier()
        pltpu.sync_copy(acc_sh.at[pl.ds(sid * zrows, zrows)],
                        out_hbm.at[cid, pl.ds(sid * zrows, zrows)])

    call = pl.kernel(
        body,
        out_type=jax.ShapeDtypeStruct((NC, ACC_PAD, D), jnp.float32),
        mesh=mesh_sc,
        scratch_types=[
            pltpu.VMEM_SHARED((ACC_PAD, D), jnp.float32),
            pltpu.VMEM((CHUNK,), jnp.int32),
            pltpu.VMEM((CHUNK,), jnp.int32),
            pltpu.VMEM((CHUNK, D), jnp.float32),
            pltpu.VMEM((64, D), jnp.float32),
        ],
    )
    return call(ef, dst_idx)


def kernel(grid_node_features, mesh_node_features, grid2mesh_edge_features,
           edge_index, halo_indices, num_local,
           eW1, eb1, eW2, eb2, eg, ebn,
           nW1, nb1, nW2, nb2, ng, nbn,
           gW1, gb1, gW2, gb2, gg, gbn):
    n_mesh = mesh_node_features.shape[0]
    dst_idx = edge_index[:, 0]
    src_idx = edge_index[:, 1]
    aug_map = jnp.concatenate([jnp.arange(n_mesh, dtype=jnp.int32),
                               halo_indices.astype(jnp.int32)])
    ws, wd, we = eW1[:D], eW1[D:2 * D], eW1[2 * D:]

    tsm, tdm = _prep(mesh_node_features, ws, wd, eb1)
    g = _sc_gather(src_idx, dst_idx, aug_map, tsm, tdm)
    ef = _edge_mlp(g, grid2mesh_edge_features, we, eW2, eb2, eg, ebn)
    partials = _sc_scatter(ef, dst_idx, n_mesh)
    mesh_out = _node_mlp(mesh_node_features, partials, nW1[:D], nW1[D:],
                         nb1, nW2, nb2, ng, nbn)
    grid_out = _resid_mlp(grid_node_features, gW1, gb1, gW2, gb2, gg, gbn,
                          blk=2000)
    return (grid_out, mesh_out)


# speedup vs baseline: 4.1542x; 1.1973x over previous
import functools

import jax
import jax.numpy as jnp
from jax import lax
from jax.experimental import pallas as pl
from jax.experimental.pallas import tpu as pltpu
from jax.experimental.pallas import tpu_sc as plsc

D = 128
CHUNK = 128
NC, NS = 2, 16
NW = NC * NS
ACC_PAD = 10240
LN_EPS = 1e-5


def _mlp2_body(x_ref, a_ref, w1x_ref, w1a_ref, b1_ref, w2_ref, b2_ref,
               g_ref, bsh_ref, out_ref):
    x = x_ref[...]
    pre = jnp.dot(x, w1x_ref[...], preferred_element_type=jnp.float32) + b1_ref[...]
    if a_ref is not None:
        pre = pre + a_ref
    h = pre * (1.0 / (1.0 + jnp.exp(-pre)))
    y = jnp.dot(h, w2_ref[...], preferred_element_type=jnp.float32) + b2_ref[...]
    mu = jnp.mean(y, axis=-1, keepdims=True)
    yc = y - mu
    var = jnp.mean(yc * yc, axis=-1, keepdims=True)
    out_ref[...] = x + yc * lax.rsqrt(var + LN_EPS) * g_ref[...] + bsh_ref[...]


def _resid_mlp(x, w1, b1, w2, b2, g, bsh, blk):
    n = x.shape[0]
    body = lambda x_ref, w1x, b1_, w2_, b2_, g_, bsh_, out_ref: _mlp2_body(
        x_ref, None, w1x, None, b1_, w2_, b2_, g_, bsh_, out_ref)
    wspec = pl.BlockSpec((D, D), lambda i: (0, 0))
    vspec = pl.BlockSpec((1, D), lambda i: (0, 0))
    return pl.pallas_call(
        body,
        grid=(n // blk,),
        in_specs=[pl.BlockSpec((blk, D), lambda i: (i, 0)),
                  wspec, vspec, wspec, vspec, vspec, vspec],
        out_specs=pl.BlockSpec((blk, D), lambda i: (i, 0)),
        out_shape=jax.ShapeDtypeStruct((n, D), jnp.float32),
    )(x, w1, b1.reshape(1, D), w2, b2.reshape(1, D), g.reshape(1, D),
      bsh.reshape(1, D))


def _prep_body(m_ref, ws_ref, wd_ref, b1_ref, ts_ref, td_ref):
    m = m_ref[...]
    ts_ref[...] = jnp.dot(m, ws_ref[...], preferred_element_type=jnp.float32) + b1_ref[...]
    td_ref[...] = jnp.dot(m, wd_ref[...], preferred_element_type=jnp.float32)


def _prep(mesh, ws, wd, b1):
    n = mesh.shape[0]
    blk = 2000
    wspec = pl.BlockSpec((D, D), lambda i: (0, 0))
    return pl.pallas_call(
        _prep_body,
        grid=(n // blk,),
        in_specs=[pl.BlockSpec((blk, D), lambda i: (i, 0)), wspec, wspec,
                  pl.BlockSpec((1, D), lambda i: (0, 0))],
        out_specs=[pl.BlockSpec((blk, D), lambda i: (i, 0))] * 2,
        out_shape=[jax.ShapeDtypeStruct((n, D), jnp.float32)] * 2,
    )(mesh, ws, wd, b1.reshape(1, D))


def _edge_body(g_ref, e_ref, we_ref, w2_ref, b2_ref, gam_ref, bet_ref, out_ref):
    e = e_ref[...]
    pre = g_ref[...] + jnp.dot(e, we_ref[...], preferred_element_type=jnp.float32)
    h = pre * (1.0 / (1.0 + jnp.exp(-pre)))
    y = jnp.dot(h, w2_ref[...], preferred_element_type=jnp.float32) + b2_ref[...]
    mu = jnp.mean(y, axis=-1, keepdims=True)
    yc = y - mu
    var = jnp.mean(yc * yc, axis=-1, keepdims=True)
    out_ref[...] = e + yc * lax.rsqrt(var + LN_EPS) * gam_ref[...] + bet_ref[...]


def _edge_mlp(g, e, we, w2, b2, gam, bet):
    n = e.shape[0]
    blk = 2000
    wspec = pl.BlockSpec((D, D), lambda i: (0, 0))
    vspec = pl.BlockSpec((1, D), lambda i: (0, 0))
    return pl.pallas_call(
        _edge_body,
        grid=(n // blk,),
        in_specs=[pl.BlockSpec((blk, D), lambda i: (i, 0)),
                  pl.BlockSpec((blk, D), lambda i: (i, 0)),
                  wspec, wspec, vspec, vspec, vspec],
        out_specs=pl.BlockSpec((blk, D), lambda i: (i, 0)),
        out_shape=jax.ShapeDtypeStruct((n, D), jnp.float32),
    )(g, e, we, w2, b2.reshape(1, D), gam.reshape(1, D), bet.reshape(1, D))


def _node_body(x_ref, p0_ref, p1_ref, w1x_ref, w1a_ref, b1_ref, w2_ref,
               b2_ref, g_ref, bsh_ref, out_ref):
    agg = p0_ref[0] + p1_ref[0]
    apre = jnp.dot(agg, w1a_ref[...], preferred_element_type=jnp.float32)
    _mlp2_body(x_ref, apre, w1x_ref, None, b1_ref, w2_ref, b2_ref, g_ref,
               bsh_ref, out_ref)


def _node_mlp(mesh, partials, w1x, w1a, b1, w2, b2, g, bsh):
    n = mesh.shape[0]
    blk = 2000
    wspec = pl.BlockSpec((D, D), lambda i: (0, 0))
    vspec = pl.BlockSpec((1, D), lambda i: (0, 0))
    return pl.pallas_call(
        _node_body,
        grid=(n // blk,),
        in_specs=[pl.BlockSpec((blk, D), lambda i: (i, 0)),
                  pl.BlockSpec((1, blk, D), lambda i: (0, i, 0)),
                  pl.BlockSpec((1, blk, D), lambda i: (1, i, 0)),
                  wspec, wspec, vspec, wspec, vspec, vspec, vspec],
        out_specs=pl.BlockSpec((blk, D), lambda i: (i, 0)),
        out_shape=jax.ShapeDtypeStruct((n, D), jnp.float32),
    )(mesh, partials, partials, w1x, w1a, b1.reshape(1, D), w2,
      b2.reshape(1, D), g.reshape(1, D), bsh.reshape(1, D))


def _sc_gather(src_idx, dst_idx, aug_map, tsm, tdm):
    n_edge = src_idx.shape[0]
    n_aug = aug_map.shape[0]
    nchunks = n_edge // CHUNK
    iters = (nchunks + NW - 1) // NW
    mesh_sc = plsc.VectorSubcoreMesh(core_axis_name="c", subcore_axis_name="s")

    def body(src_hbm, dst_hbm, map_hbm, ts_hbm, td_hbm, out_hbm,
             map_v, si_v, di_v, ms_v, md_v, rs_v, rd_v, gsem, osem):
        cid = lax.axis_index("c")
        sid = lax.axis_index("s")
        wid = sid * NC + cid
        pltpu.sync_copy(map_hbm, map_v)

        def issue(b, it):
            ci = wid + it * NW
            valid = ci < nchunks
            base = ci * CHUNK

            @pl.when(jnp.logical_and(valid, it >= 2))
            def _drain():
                pltpu.make_async_copy(rs_v.at[b],
                                      out_hbm.at[pl.ds(0, CHUNK)],
                                      osem[b]).wait()

            @pl.when(valid)
            def _():
                pltpu.sync_copy(src_hbm.at[pl.ds(base, CHUNK)], si_v.at[b])
                pltpu.sync_copy(dst_hbm.at[pl.ds(base, CHUNK)], di_v.at[b])
                for j in range(CHUNK // 16):
                    sl = pl.ds(j * 16, 16)
                    ms_v[b, sl] = plsc.load_gather(map_v, [si_v[b, sl]])
                    md_v[b, sl] = plsc.load_gather(map_v, [di_v[b, sl]])
                pltpu.async_copy(ts_hbm.at[ms_v.at[b]], rs_v.at[b], gsem[b])
                pltpu.async_copy(td_hbm.at[md_v.at[b]], rd_v.at[b], gsem[b])

        def finish(b, it):
            ci = wid + it * NW

            @pl.when(jnp.logical_and(it >= 0, ci < nchunks))
            def _():
                base = ci * CHUNK
                pltpu.make_async_copy(ts_hbm.at[ms_v.at[b]], rs_v.at[b],
                                      gsem[b]).wait()
                pltpu.make_async_copy(td_hbm.at[md_v.at[b]], rd_v.at[b],
                                      gsem[b]).wait()

                @plsc.parallel_loop(0, CHUNK, 1, unroll=4)
                def _add(r):
                    for k in range(8):
                        sl = pl.ds(k * 16, 16)
                        plsc.addupdate(rs_v.at[b, r, sl], rd_v[b, r, sl])

                pltpu.async_copy(rs_v.at[b], out_hbm.at[pl.ds(base, CHUNK)],
                                 osem[b])

        def super_body(t, carry):
            for b in range(2):
                it = 2 * t + b
                issue(b, it)
                finish(1 - b, it - 1)
            return carry

        lax.fori_loop(0, (iters + 2) // 2, super_body, 0)
        nw_chunks = lax.max((nchunks - wid + NW - 1) // NW, 0)
        for b in range(2):
            @pl.when(nw_chunks > b)
            def _():
                pltpu.make_async_copy(rs_v.at[b],
                                      out_hbm.at[pl.ds(0, CHUNK)],
                                      osem[b]).wait()

    def body_wrap(src_hbm, dst_hbm, map_hbm, ts_hbm, td_hbm, out_hbm,
                  map_v, si_v, di_v, ms_v, md_v, rs_v, rd_v,
                  gsem0, gsem1, osem0, osem1):
        return body(src_hbm, dst_hbm, map_hbm, ts_hbm, td_hbm, out_hbm,
                    map_v, si_v, di_v, ms_v, md_v, rs_v, rd_v,
                    (gsem0, gsem1), (osem0, osem1))

    call = pl.kernel(
        body_wrap,
        out_type=jax.ShapeDtypeStruct((n_edge, D), jnp.float32),
        mesh=mesh_sc,
        compiler_params=pltpu.CompilerParams(needs_layout_passes=False),
        scratch_types=[
            pltpu.VMEM((n_aug,), jnp.int32),
            pltpu.VMEM((2, CHUNK), jnp.int32),
            pltpu.VMEM((2, CHUNK), jnp.int32),
            pltpu.VMEM((2, CHUNK), jnp.int32),
            pltpu.VMEM((2, CHUNK), jnp.int32),
            pltpu.VMEM((2, CHUNK, D), jnp.float32),
            pltpu.VMEM((2, CHUNK, D), jnp.float32),
            pltpu.SemaphoreType.DMA,
            pltpu.SemaphoreType.DMA,
            pltpu.SemaphoreType.DMA,
            pltpu.SemaphoreType.DMA,
        ],
    )
    return call(src_idx, dst_idx, aug_map, tsm, tdm)


def _sc_scatter(ef, dst_idx, n_mesh):
    n_edge = ef.shape[0]
    nchunks = n_edge // CHUNK
    iters = (nchunks + NW - 1) // NW
    zrows = ACC_PAD // NS
    mesh_sc = plsc.VectorSubcoreMesh(core_axis_name="c", subcore_axis_name="s")

    def body(ef_hbm, di_hbm, out_hbm, acc_sh, ids_v, mp_v, rows_v, zb_v):
        cid = lax.axis_index("c")
        sid = lax.axis_index("s")
        wid = sid * NC + cid

        @plsc.parallel_loop(0, 64, 1, unroll=4)
        def _zb(r):
            for k in range(8):
                zb_v[r, pl.ds(k * 16, 16)] = jnp.zeros((16,), jnp.float32)

        def zc(i, c):
            pltpu.sync_copy(zb_v, acc_sh.at[pl.ds(sid * zrows + i * 64, 64)])
            return c

        lax.fori_loop(0, zrows // 64, zc, 0)
        plsc.subcore_barrier()

        def chunk_body(it, carry):
            ci = wid + it * NW

            @pl.when(ci < nchunks)
            def _():
                base = ci * CHUNK
                pltpu.sync_copy(di_hbm.at[pl.ds(base, CHUNK)], ids_v)
                pltpu.sync_copy(ef_hbm.at[pl.ds(base, CHUNK)], rows_v)
                for j in range(CHUNK // 16):
                    sl = pl.ds(j * 16, 16)
                    v = ids_v[sl]
                    mp_v[sl] = jnp.where(v < n_mesh, v, n_mesh)
                pltpu.sync_copy(rows_v, acc_sh.at[mp_v], add=True)

            return carry

        lax.fori_loop(0, iters, chunk_body, 0)
        plsc.subcore_barrier()
        pltpu.sync_copy(acc_sh.at[pl.ds(sid * zrows, zrows)],
                        out_hbm.at[cid, pl.ds(sid * zrows, zrows)])

    call = pl.kernel(
        body,
        out_type=jax.ShapeDtypeStruct((NC, ACC_PAD, D), jnp.float32),
        mesh=mesh_sc,
        scratch_types=[
            pltpu.VMEM_SHARED((ACC_PAD, D), jnp.float32),
            pltpu.VMEM((CHUNK,), jnp.int32),
            pltpu.VMEM((CHUNK,), jnp.int32),
            pltpu.VMEM((CHUNK, D), jnp.float32),
            pltpu.VMEM((64, D), jnp.float32),
        ],
    )
    return call(ef, dst_idx)


def kernel(grid_node_features, mesh_node_features, grid2mesh_edge_features,
           edge_index, halo_indices, num_local,
           eW1, eb1, eW2, eb2, eg, ebn,
           nW1, nb1, nW2, nb2, ng, nbn,
           gW1, gb1, gW2, gb2, gg, gbn):
    n_mesh = mesh_node_features.shape[0]
    dst_idx = edge_index[:, 0]
    src_idx = edge_index[:, 1]
    aug_map = jnp.concatenate([jnp.arange(n_mesh, dtype=jnp.int32),
                               halo_indices.astype(jnp.int32)])
    ws, wd, we = eW1[:D], eW1[D:2 * D], eW1[2 * D:]

    tsm, tdm = _prep(mesh_node_features, ws, wd, eb1)
    g = _sc_gather(src_idx, dst_idx, aug_map, tsm, tdm)
    ef = _edge_mlp(g, grid2mesh_edge_features, we, eW2, eb2, eg, ebn)
    partials = _sc_scatter(ef, dst_idx, n_mesh)
    mesh_out = _node_mlp(mesh_node_features, partials, nW1[:D], nW1[D:],
                         nb1, nW2, nb2, ng, nbn)
    grid_out = _resid_mlp(grid_node_features, gW1, gb1, gW2, gb2, gg, gbn,
                          blk=2000)
    return (grid_out, mesh_out)


# speedup vs baseline: 4.7885x; 1.1527x over previous
import functools

import jax
import jax.numpy as jnp
from jax import lax
from jax.experimental import pallas as pl
from jax.experimental.pallas import tpu as pltpu
from jax.experimental.pallas import tpu_sc as plsc

D = 128
CHUNK = 128
NC, NS = 2, 16
NW = NC * NS
ACC_PAD = 10240
LN_EPS = 1e-5


def _mlp2_body(x_ref, a_ref, w1x_ref, w1a_ref, b1_ref, w2_ref, b2_ref,
               g_ref, bsh_ref, out_ref):
    x = x_ref[...]
    pre = jnp.dot(x, w1x_ref[...], preferred_element_type=jnp.float32) + b1_ref[...]
    if a_ref is not None:
        pre = pre + a_ref
    h = pre * (1.0 / (1.0 + jnp.exp(-pre)))
    y = jnp.dot(h, w2_ref[...], preferred_element_type=jnp.float32) + b2_ref[...]
    mu = jnp.mean(y, axis=-1, keepdims=True)
    yc = y - mu
    var = jnp.mean(yc * yc, axis=-1, keepdims=True)
    out_ref[...] = x + yc * lax.rsqrt(var + LN_EPS) * g_ref[...] + bsh_ref[...]


def _resid_mlp(x, w1, b1, w2, b2, g, bsh, blk):
    n = x.shape[0]
    body = lambda x_ref, w1x, b1_, w2_, b2_, g_, bsh_, out_ref: _mlp2_body(
        x_ref, None, w1x, None, b1_, w2_, b2_, g_, bsh_, out_ref)
    wspec = pl.BlockSpec((D, D), lambda i: (0, 0))
    vspec = pl.BlockSpec((1, D), lambda i: (0, 0))
    return pl.pallas_call(
        body,
        grid=(n // blk,),
        in_specs=[pl.BlockSpec((blk, D), lambda i: (i, 0)),
                  wspec, vspec, wspec, vspec, vspec, vspec],
        out_specs=pl.BlockSpec((blk, D), lambda i: (i, 0)),
        out_shape=jax.ShapeDtypeStruct((n, D), jnp.float32),
    )(x, w1, b1.reshape(1, D), w2, b2.reshape(1, D), g.reshape(1, D),
      bsh.reshape(1, D))


def _prep_body(m_ref, ws_ref, wd_ref, b1_ref, ts_ref, td_ref):
    m = m_ref[...]
    ts_ref[...] = jnp.dot(m, ws_ref[...], preferred_element_type=jnp.float32) + b1_ref[...]
    td_ref[...] = jnp.dot(m, wd_ref[...], preferred_element_type=jnp.float32)


def _prep(mesh, ws, wd, b1):
    n = mesh.shape[0]
    blk = 2000
    wspec = pl.BlockSpec((D, D), lambda i: (0, 0))
    return pl.pallas_call(
        _prep_body,
        grid=(n // blk,),
        in_specs=[pl.BlockSpec((blk, D), lambda i: (i, 0)), wspec, wspec,
                  pl.BlockSpec((1, D), lambda i: (0, 0))],
        out_specs=[pl.BlockSpec((blk, D), lambda i: (i, 0))] * 2,
        out_shape=[jax.ShapeDtypeStruct((n, D), jnp.float32)] * 2,
    )(mesh, ws, wd, b1.reshape(1, D))


def _edge_body(g_ref, e_ref, we_ref, w2_ref, b2_ref, gam_ref, bet_ref, out_ref):
    e = e_ref[...]
    pre = g_ref[...] + jnp.dot(e, we_ref[...], preferred_element_type=jnp.float32)
    h = pre * (1.0 / (1.0 + jnp.exp(-pre)))
    y = jnp.dot(h, w2_ref[...], preferred_element_type=jnp.float32) + b2_ref[...]
    mu = jnp.mean(y, axis=-1, keepdims=True)
    yc = y - mu
    var = jnp.mean(yc * yc, axis=-1, keepdims=True)
    out_ref[...] = e + yc * lax.rsqrt(var + LN_EPS) * gam_ref[...] + bet_ref[...]


def _edge_mlp(g, e, we, w2, b2, gam, bet):
    n = e.shape[0]
    blk = 2000
    wspec = pl.BlockSpec((D, D), lambda i: (0, 0))
    vspec = pl.BlockSpec((1, D), lambda i: (0, 0))
    return pl.pallas_call(
        _edge_body,
        grid=(n // blk,),
        in_specs=[pl.BlockSpec((blk, D), lambda i: (i, 0)),
                  pl.BlockSpec((blk, D), lambda i: (i, 0)),
                  wspec, wspec, vspec, vspec, vspec],
        out_specs=pl.BlockSpec((blk, D), lambda i: (i, 0)),
        out_shape=jax.ShapeDtypeStruct((n, D), jnp.float32),
    )(g, e, we, w2, b2.reshape(1, D), gam.reshape(1, D), bet.reshape(1, D))


def _node_body(x_ref, p0_ref, p1_ref, w1x_ref, w1a_ref, b1_ref, w2_ref,
               b2_ref, g_ref, bsh_ref, out_ref):
    agg = p0_ref[0] + p1_ref[0]
    apre = jnp.dot(agg, w1a_ref[...], preferred_element_type=jnp.float32)
    _mlp2_body(x_ref, apre, w1x_ref, None, b1_ref, w2_ref, b2_ref, g_ref,
               bsh_ref, out_ref)


def _node_mlp(mesh, partials, w1x, w1a, b1, w2, b2, g, bsh):
    n = mesh.shape[0]
    blk = 2000
    wspec = pl.BlockSpec((D, D), lambda i: (0, 0))
    vspec = pl.BlockSpec((1, D), lambda i: (0, 0))
    return pl.pallas_call(
        _node_body,
        grid=(n // blk,),
        in_specs=[pl.BlockSpec((blk, D), lambda i: (i, 0)),
                  pl.BlockSpec((1, blk, D), lambda i: (0, i, 0)),
                  pl.BlockSpec((1, blk, D), lambda i: (1, i, 0)),
                  wspec, wspec, vspec, wspec, vspec, vspec, vspec],
        out_specs=pl.BlockSpec((blk, D), lambda i: (i, 0)),
        out_shape=jax.ShapeDtypeStruct((n, D), jnp.float32),
    )(mesh, partials, partials, w1x, w1a, b1.reshape(1, D), w2,
      b2.reshape(1, D), g.reshape(1, D), bsh.reshape(1, D))


def _sc_gather(src_idx, dst_idx, aug_map, tsm, tdm):
    n_edge = src_idx.shape[0]
    n_aug = aug_map.shape[0]
    nchunks = n_edge // CHUNK
    iters = (nchunks + NW - 1) // NW
    mesh_sc = plsc.VectorSubcoreMesh(core_axis_name="c", subcore_axis_name="s")

    def body(src_hbm, dst_hbm, map_hbm, ts_hbm, td_hbm, out_hbm,
             map_v, si_v, di_v, ms_v, md_v, rs_v, rd_v, gsem, osem):
        cid = lax.axis_index("c")
        sid = lax.axis_index("s")
        wid = sid * NC + cid
        pltpu.sync_copy(map_hbm, map_v)

        def issue(b, it):
            ci = wid + it * NW
            valid = ci < nchunks
            base = ci * CHUNK

            @pl.when(jnp.logical_and(valid, it >= 2))
            def _drain():
                pltpu.make_async_copy(rs_v.at[b],
                                      out_hbm.at[pl.ds(0, CHUNK)],
                                      osem[b]).wait()

            @pl.when(valid)
            def _():
                pltpu.sync_copy(src_hbm.at[pl.ds(base, CHUNK)], si_v.at[b])
                pltpu.sync_copy(dst_hbm.at[pl.ds(base, CHUNK)], di_v.at[b])
                for j in range(CHUNK // 16):
                    sl = pl.ds(j * 16, 16)
                    ms_v[b, sl] = plsc.load_gather(map_v, [si_v[b, sl]])
                    md_v[b, sl] = plsc.load_gather(map_v, [di_v[b, sl]])
                pltpu.async_copy(ts_hbm.at[ms_v.at[b]], rs_v.at[b], gsem[b])
                pltpu.async_copy(td_hbm.at[md_v.at[b]], rd_v.at[b], gsem[b])

        def finish(b, it):
            ci = wid + it * NW

            @pl.when(jnp.logical_and(it >= 0, ci < nchunks))
            def _():
                base = ci * CHUNK
                pltpu.make_async_copy(ts_hbm.at[ms_v.at[b]], rs_v.at[b],
                                      gsem[b]).wait()
                pltpu.make_async_copy(td_hbm.at[md_v.at[b]], rd_v.at[b],
                                      gsem[b]).wait()

                @plsc.parallel_loop(0, CHUNK, 1, unroll=4)
                def _add(r):
                    for k in range(8):
                        sl = pl.ds(k * 16, 16)
                        plsc.addupdate(rs_v.at[b, r, sl], rd_v[b, r, sl])

                pltpu.async_copy(rs_v.at[b], out_hbm.at[pl.ds(base, CHUNK)],
                                 osem[b])

        def super_body(t, carry):
            for b in range(2):
                it = 2 * t + b
                issue(b, it)
                finish(1 - b, it - 1)
            return carry

        lax.fori_loop(0, (iters + 2) // 2, super_body, 0)
        nw_chunks = lax.max((nchunks - wid + NW - 1) // NW, 0)
        for b in range(2):
            @pl.when(nw_chunks > b)
            def _():
                pltpu.make_async_copy(rs_v.at[b],
                                      out_hbm.at[pl.ds(0, CHUNK)],
                                      osem[b]).wait()

    def body_wrap(src_hbm, dst_hbm, map_hbm, ts_hbm, td_hbm, out_hbm,
                  map_v, si_v, di_v, ms_v, md_v, rs_v, rd_v,
                  gsem0, gsem1, osem0, osem1):
        return body(src_hbm, dst_hbm, map_hbm, ts_hbm, td_hbm, out_hbm,
                    map_v, si_v, di_v, ms_v, md_v, rs_v, rd_v,
                    (gsem0, gsem1), (osem0, osem1))

    call = pl.kernel(
        body_wrap,
        out_type=jax.ShapeDtypeStruct((n_edge, D), jnp.float32),
        mesh=mesh_sc,
        compiler_params=pltpu.CompilerParams(needs_layout_passes=False),
        scratch_types=[
            pltpu.VMEM((n_aug,), jnp.int32),
            pltpu.VMEM((2, CHUNK), jnp.int32),
            pltpu.VMEM((2, CHUNK), jnp.int32),
            pltpu.VMEM((2, CHUNK), jnp.int32),
            pltpu.VMEM((2, CHUNK), jnp.int32),
            pltpu.VMEM((2, CHUNK, D), jnp.float32),
            pltpu.VMEM((2, CHUNK, D), jnp.float32),
            pltpu.SemaphoreType.DMA,
            pltpu.SemaphoreType.DMA,
            pltpu.SemaphoreType.DMA,
            pltpu.SemaphoreType.DMA,
        ],
    )
    return call(src_idx, dst_idx, aug_map, tsm, tdm)


def _sc_scatter(ef, dst_idx, n_mesh):
    n_edge = ef.shape[0]
    nchunks = n_edge // CHUNK
    iters = (nchunks + NW - 1) // NW
    zrows = ACC_PAD // NS
    mesh_sc = plsc.VectorSubcoreMesh(core_axis_name="c", subcore_axis_name="s")

    def body(ef_hbm, di_hbm, out_hbm, acc_sh, ids_v, mp_v, rows_v, zb_v,
             lsem0, lsem1, ssem0, ssem1):
        cid = lax.axis_index("c")
        sid = lax.axis_index("s")
        wid = sid * NC + cid
        lsem = (lsem0, lsem1)
        ssem = (ssem0, ssem1)

        @plsc.parallel_loop(0, 64, 1, unroll=4)
        def _zb(r):
            for k in range(8):
                zb_v[r, pl.ds(k * 16, 16)] = jnp.zeros((16,), jnp.float32)

        def zc(i, c):
            pltpu.sync_copy(zb_v, acc_sh.at[pl.ds(sid * zrows + i * 64, 64)])
            return c

        lax.fori_loop(0, zrows // 64, zc, 0)
        plsc.subcore_barrier()

        def load(b, it):
            ci = wid + it * NW
            valid = ci < nchunks
            base = ci * CHUNK

            @pl.when(jnp.logical_and(valid, it >= 2))
            def _drain():
                pltpu.make_async_copy(rows_v.at[b], acc_sh.at[mp_v.at[b]],
                                      ssem[b]).wait()

            @pl.when(valid)
            def _():
                pltpu.async_copy(di_hbm.at[pl.ds(base, CHUNK)], ids_v.at[b],
                                 lsem[b])
                pltpu.async_copy(ef_hbm.at[pl.ds(base, CHUNK)], rows_v.at[b],
                                 lsem[b])

        def process(b, it):
            ci = wid + it * NW

            @pl.when(jnp.logical_and(it >= 0, ci < nchunks))
            def _():
                base = ci * CHUNK
                pltpu.make_async_copy(di_hbm.at[pl.ds(base, CHUNK)],
                                      ids_v.at[b], lsem[b]).wait()
                pltpu.make_async_copy(ef_hbm.at[pl.ds(base, CHUNK)],
                                      rows_v.at[b], lsem[b]).wait()
                for j in range(CHUNK // 16):
                    sl = pl.ds(j * 16, 16)
                    v = ids_v[b, sl]
                    mp_v[b, sl] = jnp.where(v < n_mesh, v, n_mesh)
                pltpu.async_copy(rows_v.at[b], acc_sh.at[mp_v.at[b]],
                                 ssem[b], add=True)

        def super_body(t, carry):
            for b in range(2):
                it = 2 * t + b
                load(b, it)
                process(1 - b, it - 1)
            return carry

        lax.fori_loop(0, (iters + 2) // 2, super_body, 0)
        nw_chunks = lax.max((nchunks - wid + NW - 1) // NW, 0)
        for b in range(2):
            @pl.when(nw_chunks > b)
            def _():
                pltpu.make_async_copy(rows_v.at[b], acc_sh.at[mp_v.at[b]],
                                      ssem[b]).wait()
        plsc.subcore_barrier()
        pltpu.sync_copy(acc_sh.at[pl.ds(sid * zrows, zrows)],
                        out_hbm.at[cid, pl.ds(sid * zrows, zrows)])

    call = pl.kernel(
        body,
        out_type=jax.ShapeDtypeStruct((NC, ACC_PAD, D), jnp.float32),
        mesh=mesh_sc,
        compiler_params=pltpu.CompilerParams(needs_layout_passes=False),
        scratch_types=[
            pltpu.VMEM_SHARED((ACC_PAD, D), jnp.float32),
            pltpu.VMEM((2, CHUNK), jnp.int32),
            pltpu.VMEM((2, CHUNK), jnp.int32),
            pltpu.VMEM((2, CHUNK, D), jnp.float32),
            pltpu.VMEM((64, D), jnp.float32),
            pltpu.SemaphoreType.DMA,
            pltpu.SemaphoreType.DMA,
            pltpu.SemaphoreType.DMA,
            pltpu.SemaphoreType.DMA,
        ],
    )
    return call(ef, dst_idx)


def kernel(grid_node_features, mesh_node_features, grid2mesh_edge_features,
           edge_index, halo_indices, num_local,
           eW1, eb1, eW2, eb2, eg, ebn,
           nW1, nb1, nW2, nb2, ng, nbn,
           gW1, gb1, gW2, gb2, gg, gbn):
    n_mesh = mesh_node_features.shape[0]
    dst_idx = edge_index[:, 0]
    src_idx = edge_index[:, 1]
    aug_map = jnp.concatenate([jnp.arange(n_mesh, dtype=jnp.int32),
                               halo_indices.astype(jnp.int32)])
    ws, wd, we = eW1[:D], eW1[D:2 * D], eW1[2 * D:]

    tsm, tdm = _prep(mesh_node_features, ws, wd, eb1)
    g = _sc_gather(src_idx, dst_idx, aug_map, tsm, tdm)
    ef = _edge_mlp(g, grid2mesh_edge_features, we, eW2, eb2, eg, ebn)
    partials = _sc_scatter(ef, dst_idx, n_mesh)
    mesh_out = _node_mlp(mesh_node_features, partials, nW1[:D], nW1[D:],
                         nb1, nW2, nb2, ng, nbn)
    grid_out = _resid_mlp(grid_node_features, gW1, gb1, gW2, gb2, gg, gbn,
                          blk=2000)
    return (grid_out, mesh_out)


# speedup vs baseline: 5.0828x; 1.0614x over previous
import functools

import jax
import jax.numpy as jnp
from jax import lax
from jax.experimental import pallas as pl
from jax.experimental.pallas import tpu as pltpu
from jax.experimental.pallas import tpu_sc as plsc

D = 128
CHUNK = 128
NC, NS = 2, 16
NW = NC * NS
ACC_PAD = 10240
LN_EPS = 1e-5


def _mlp2_body(x_ref, a_ref, w1x_ref, w1a_ref, b1_ref, w2_ref, b2_ref,
               g_ref, bsh_ref, out_ref):
    x = x_ref[...]
    pre = jnp.dot(x, w1x_ref[...], preferred_element_type=jnp.float32) + b1_ref[...]
    if a_ref is not None:
        pre = pre + a_ref
    h = pre * (1.0 / (1.0 + jnp.exp(-pre)))
    y = jnp.dot(h, w2_ref[...], preferred_element_type=jnp.float32) + b2_ref[...]
    mu = jnp.mean(y, axis=-1, keepdims=True)
    yc = y - mu
    var = jnp.mean(yc * yc, axis=-1, keepdims=True)
    out_ref[...] = x + yc * lax.rsqrt(var + LN_EPS) * g_ref[...] + bsh_ref[...]


def _resid_mlp(x, w1, b1, w2, b2, g, bsh, blk):
    n = x.shape[0]
    body = lambda x_ref, w1x, b1_, w2_, b2_, g_, bsh_, out_ref: _mlp2_body(
        x_ref, None, w1x, None, b1_, w2_, b2_, g_, bsh_, out_ref)
    wspec = pl.BlockSpec((D, D), lambda i: (0, 0))
    vspec = pl.BlockSpec((1, D), lambda i: (0, 0))
    return pl.pallas_call(
        body,
        grid=(n // blk,),
        in_specs=[pl.BlockSpec((blk, D), lambda i: (i, 0)),
                  wspec, vspec, wspec, vspec, vspec, vspec],
        out_specs=pl.BlockSpec((blk, D), lambda i: (i, 0)),
        out_shape=jax.ShapeDtypeStruct((n, D), jnp.float32),
    )(x, w1, b1.reshape(1, D), w2, b2.reshape(1, D), g.reshape(1, D),
      bsh.reshape(1, D))


def _prep_body(m_ref, ws_ref, wd_ref, b1_ref, ts_ref, td_ref):
    m = m_ref[...]
    ts_ref[...] = jnp.dot(m, ws_ref[...], preferred_element_type=jnp.float32) + b1_ref[...]
    td_ref[...] = jnp.dot(m, wd_ref[...], preferred_element_type=jnp.float32)


def _prep(mesh, ws, wd, b1):
    n = mesh.shape[0]
    blk = 2000
    wspec = pl.BlockSpec((D, D), lambda i: (0, 0))
    return pl.pallas_call(
        _prep_body,
        grid=(n // blk,),
        in_specs=[pl.BlockSpec((blk, D), lambda i: (i, 0)), wspec, wspec,
                  pl.BlockSpec((1, D), lambda i: (0, 0))],
        out_specs=[pl.BlockSpec((blk, D), lambda i: (i, 0))] * 2,
        out_shape=[jax.ShapeDtypeStruct((n, D), jnp.float32)] * 2,
    )(mesh, ws, wd, b1.reshape(1, D))


def _edge_body(g_ref, e_ref, we_ref, w2_ref, b2_ref, gam_ref, bet_ref, out_ref):
    e = e_ref[...]
    pre = g_ref[...] + jnp.dot(e, we_ref[...], preferred_element_type=jnp.float32)
    h = pre * (1.0 / (1.0 + jnp.exp(-pre)))
    y = jnp.dot(h, w2_ref[...], preferred_element_type=jnp.float32) + b2_ref[...]
    mu = jnp.mean(y, axis=-1, keepdims=True)
    yc = y - mu
    var = jnp.mean(yc * yc, axis=-1, keepdims=True)
    out_ref[...] = e + yc * lax.rsqrt(var + LN_EPS) * gam_ref[...] + bet_ref[...]


def _edge_mlp(g, e, we, w2, b2, gam, bet):
    n = e.shape[0]
    blk = 2000
    wspec = pl.BlockSpec((D, D), lambda i: (0, 0))
    vspec = pl.BlockSpec((1, D), lambda i: (0, 0))
    return pl.pallas_call(
        _edge_body,
        grid=(n // blk,),
        in_specs=[pl.BlockSpec((blk, D), lambda i: (i, 0)),
                  pl.BlockSpec((blk, D), lambda i: (i, 0)),
                  wspec, wspec, vspec, vspec, vspec],
        out_specs=pl.BlockSpec((blk, D), lambda i: (i, 0)),
        out_shape=jax.ShapeDtypeStruct((n, D), jnp.float32),
    )(g, e, we, w2, b2.reshape(1, D), gam.reshape(1, D), bet.reshape(1, D))


def _node_mlp(mesh, partials, w1x, w1a, b1, w2, b2, g, bsh):
    n = mesh.shape[0]
    blk = 2000
    nplanes = len(partials) * NC
    wspec = pl.BlockSpec((D, D), lambda i: (0, 0))
    vspec = pl.BlockSpec((1, D), lambda i: (0, 0))

    def body(*refs):
        x_ref = refs[0]
        p_refs = refs[1:1 + nplanes]
        w1x_ref, w1a_ref, b1_ref, w2_ref, b2_ref, g_ref, bsh_ref = \
            refs[1 + nplanes:-1]
        out_ref = refs[-1]
        agg = p_refs[0][0]
        for p in p_refs[1:]:
            agg = agg + p[0]
        apre = jnp.dot(agg, w1a_ref[...], preferred_element_type=jnp.float32)
        _mlp2_body(x_ref, apre, w1x_ref, None, b1_ref, w2_ref, b2_ref, g_ref,
                   bsh_ref, out_ref)

    pspecs = []
    pargs = []
    for part in partials:
        for c in range(NC):
            pspecs.append(pl.BlockSpec((1, blk, D), lambda i, c=c: (c, i, 0)))
            pargs.append(part)
    return pl.pallas_call(
        body,
        grid=(n // blk,),
        in_specs=[pl.BlockSpec((blk, D), lambda i: (i, 0))] + pspecs +
                 [wspec, wspec, vspec, wspec, vspec, vspec, vspec],
        out_specs=pl.BlockSpec((blk, D), lambda i: (i, 0)),
        out_shape=jax.ShapeDtypeStruct((n, D), jnp.float32),
    )(mesh, *pargs, w1x, w1a, b1.reshape(1, D), w2,
      b2.reshape(1, D), g.reshape(1, D), bsh.reshape(1, D))


def _sc_gather(src_idx, dst_idx, aug_map, tsm, tdm):
    n_edge = src_idx.shape[0]
    n_aug = aug_map.shape[0]
    nchunks = n_edge // CHUNK
    iters = (nchunks + NW - 1) // NW
    mesh_sc = plsc.VectorSubcoreMesh(core_axis_name="c", subcore_axis_name="s")

    def body(src_hbm, dst_hbm, map_hbm, ts_hbm, td_hbm, out_hbm,
             map_v, si_v, di_v, ms_v, md_v, rs_v, rd_v, gsem, osem):
        cid = lax.axis_index("c")
        sid = lax.axis_index("s")
        wid = sid * NC + cid
        pltpu.sync_copy(map_hbm, map_v)

        def issue(b, it):
            ci = wid + it * NW
            valid = ci < nchunks
            base = ci * CHUNK

            @pl.when(jnp.logical_and(valid, it >= 2))
            def _drain():
                pltpu.make_async_copy(rs_v.at[b],
                                      out_hbm.at[pl.ds(0, CHUNK)],
                                      osem[b]).wait()

            @pl.when(valid)
            def _():
                pltpu.sync_copy(src_hbm.at[pl.ds(base, CHUNK)], si_v.at[b])
                pltpu.sync_copy(dst_hbm.at[pl.ds(base, CHUNK)], di_v.at[b])
                for j in range(CHUNK // 16):
                    sl = pl.ds(j * 16, 16)
                    ms_v[b, sl] = plsc.load_gather(map_v, [si_v[b, sl]])
                    md_v[b, sl] = plsc.load_gather(map_v, [di_v[b, sl]])
                pltpu.async_copy(ts_hbm.at[ms_v.at[b]], rs_v.at[b], gsem[b])
                pltpu.async_copy(td_hbm.at[md_v.at[b]], rd_v.at[b], gsem[b])

        def finish(b, it):
            ci = wid + it * NW

            @pl.when(jnp.logical_and(it >= 0, ci < nchunks))
            def _():
                base = ci * CHUNK
                pltpu.make_async_copy(ts_hbm.at[ms_v.at[b]], rs_v.at[b],
                                      gsem[b]).wait()
                pltpu.make_async_copy(td_hbm.at[md_v.at[b]], rd_v.at[b],
                                      gsem[b]).wait()

                @plsc.parallel_loop(0, CHUNK, 1, unroll=4)
                def _add(r):
                    for k in range(8):
                        sl = pl.ds(k * 16, 16)
                        plsc.addupdate(rs_v.at[b, r, sl], rd_v[b, r, sl])

                pltpu.async_copy(rs_v.at[b], out_hbm.at[pl.ds(base, CHUNK)],
                                 osem[b])

        def super_body(t, carry):
            for b in range(2):
                it = 2 * t + b
                issue(b, it)
                finish(1 - b, it - 1)
            return carry

        lax.fori_loop(0, (iters + 2) // 2, super_body, 0)
        nw_chunks = lax.max((nchunks - wid + NW - 1) // NW, 0)
        for b in range(2):
            @pl.when(nw_chunks > b)
            def _():
                pltpu.make_async_copy(rs_v.at[b],
                                      out_hbm.at[pl.ds(0, CHUNK)],
                                      osem[b]).wait()

    def body_wrap(src_hbm, dst_hbm, map_hbm, ts_hbm, td_hbm, out_hbm,
                  map_v, si_v, di_v, ms_v, md_v, rs_v, rd_v,
                  gsem0, gsem1, osem0, osem1):
        return body(src_hbm, dst_hbm, map_hbm, ts_hbm, td_hbm, out_hbm,
                    map_v, si_v, di_v, ms_v, md_v, rs_v, rd_v,
                    (gsem0, gsem1), (osem0, osem1))

    call = pl.kernel(
        body_wrap,
        out_type=jax.ShapeDtypeStruct((n_edge, D), jnp.float32),
        mesh=mesh_sc,
        compiler_params=pltpu.CompilerParams(needs_layout_passes=False),
        scratch_types=[
            pltpu.VMEM((n_aug,), jnp.int32),
            pltpu.VMEM((2, CHUNK), jnp.int32),
            pltpu.VMEM((2, CHUNK), jnp.int32),
            pltpu.VMEM((2, CHUNK), jnp.int32),
            pltpu.VMEM((2, CHUNK), jnp.int32),
            pltpu.VMEM((2, CHUNK, D), jnp.float32),
            pltpu.VMEM((2, CHUNK, D), jnp.float32),
            pltpu.SemaphoreType.DMA,
            pltpu.SemaphoreType.DMA,
            pltpu.SemaphoreType.DMA,
            pltpu.SemaphoreType.DMA,
        ],
    )
    return call(src_idx, dst_idx, aug_map, tsm, tdm)


def _sc_scatter(ef_parts, dst_parts, n_mesh):
    nparts = len(ef_parts)
    n_edge = ef_parts[0].shape[0]
    nchunks = n_edge // CHUNK
    iters = (nchunks + NW - 1) // NW
    zrows = ACC_PAD // NS
    mesh_sc = plsc.VectorSubcoreMesh(core_axis_name="c", subcore_axis_name="s")

    def body(*refs):
        ef_hbm = refs[:nparts]
        di_hbm = refs[nparts:2 * nparts]
        out_hbm = refs[2 * nparts]
        (acc_sh, ids_v, mp_v, rows_v, zb_v,
         lsem0, lsem1, ssem0, ssem1) = refs[2 * nparts + 1:]
        cid = lax.axis_index("c")
        sid = lax.axis_index("s")
        wid = sid * NC + cid
        lsem = (lsem0, lsem1)
        ssem = (ssem0, ssem1)

        @plsc.parallel_loop(0, 64, 1, unroll=4)
        def _zb(r):
            for k in range(8):
                zb_v[r, pl.ds(k * 16, 16)] = jnp.zeros((16,), jnp.float32)

        def zc(i, c):
            pltpu.sync_copy(zb_v, acc_sh.at[pl.ds(sid * zrows + i * 64, 64)])
            return c

        lax.fori_loop(0, zrows // 64, zc, 0)
        plsc.subcore_barrier()

        def load(p, b, it, drain):
            ci = wid + it * NW
            valid = ci < nchunks
            base = ci * CHUNK

            @pl.when(jnp.logical_and(valid, drain))
            def _drain():
                pltpu.make_async_copy(rows_v.at[b], acc_sh.at[mp_v.at[b]],
                                      ssem[b]).wait()

            @pl.when(valid)
            def _():
                pltpu.async_copy(di_hbm[p].at[pl.ds(base, CHUNK)],
                                 ids_v.at[b], lsem[b])
                pltpu.async_copy(ef_hbm[p].at[pl.ds(base, CHUNK)],
                                 rows_v.at[b], lsem[b])

        def process(p, b, it):
            ci = wid + it * NW

            @pl.when(jnp.logical_and(it >= 0, ci < nchunks))
            def _():
                base = ci * CHUNK
                pltpu.make_async_copy(di_hbm[p].at[pl.ds(base, CHUNK)],
                                      ids_v.at[b], lsem[b]).wait()
                pltpu.make_async_copy(ef_hbm[p].at[pl.ds(base, CHUNK)],
                                      rows_v.at[b], lsem[b]).wait()
                for j in range(CHUNK // 16):
                    sl = pl.ds(j * 16, 16)
                    v = ids_v[b, sl]
                    mp_v[b, sl] = jnp.where(v < n_mesh, v, n_mesh)
                pltpu.async_copy(rows_v.at[b], acc_sh.at[mp_v.at[b]],
                                 ssem[b], add=True)

        nw_chunks = lax.max((nchunks - wid + NW - 1) // NW, 0)
        for p in range(nparts):
            def super_body(t, carry, p=p):
                for b in range(2):
                    it = 2 * t + b
                    drain = (it >= 2) if p == 0 else (it >= 0)
                    load(p, b, it, drain)
                    process(p, 1 - b, it - 1)
                return carry

            lax.fori_loop(0, (iters + 2) // 2, super_body, 0)
        for b in range(2):
            @pl.when(nw_chunks > b)
            def _():
                pltpu.make_async_copy(rows_v.at[b], acc_sh.at[mp_v.at[b]],
                                      ssem[b]).wait()
        plsc.subcore_barrier()
        pltpu.sync_copy(acc_sh.at[pl.ds(sid * zrows, zrows)],
                        out_hbm.at[cid, pl.ds(sid * zrows, zrows)])

    call = pl.kernel(
        body,
        out_type=jax.ShapeDtypeStruct((NC, ACC_PAD, D), jnp.float32),
        mesh=mesh_sc,
        compiler_params=pltpu.CompilerParams(needs_layout_passes=False),
        scratch_types=[
            pltpu.VMEM_SHARED((ACC_PAD, D), jnp.float32),
            pltpu.VMEM((2, CHUNK), jnp.int32),
            pltpu.VMEM((2, CHUNK), jnp.int32),
            pltpu.VMEM((2, CHUNK, D), jnp.float32),
            pltpu.VMEM((64, D), jnp.float32),
            pltpu.SemaphoreType.DMA,
            pltpu.SemaphoreType.DMA,
            pltpu.SemaphoreType.DMA,
            pltpu.SemaphoreType.DMA,
        ],
    )
    return call(*ef_parts, *dst_parts)


def kernel(grid_node_features, mesh_node_features, grid2mesh_edge_features,
           edge_index, halo_indices, num_local,
           eW1, eb1, eW2, eb2, eg, ebn,
           nW1, nb1, nW2, nb2, ng, nbn,
           gW1, gb1, gW2, gb2, gg, gbn):
    n_mesh = mesh_node_features.shape[0]
    dst_idx = edge_index[:, 0]
    src_idx = edge_index[:, 1]
    aug_map = jnp.concatenate([jnp.arange(n_mesh, dtype=jnp.int32),
                               halo_indices.astype(jnp.int32)])
    ws, wd, we = eW1[:D], eW1[D:2 * D], eW1[2 * D:]

    tsm, tdm = _prep(mesh_node_features, ws, wd, eb1)

    nparts = 4
    n_edge = src_idx.shape[0]
    part = n_edge // nparts
    ef_parts, dst_parts = [], []
    for k in range(nparts):
        sl = slice(k * part, (k + 1) * part)
        src_k, dst_k = src_idx[sl], dst_idx[sl]
        g_k = _sc_gather(src_k, dst_k, aug_map, tsm, tdm)
        ef_parts.append(_edge_mlp(g_k, grid2mesh_edge_features[sl], we, eW2,
                                  eb2, eg, ebn))
        dst_parts.append(dst_k)
    partials = [
        _sc_scatter(ef_parts[:2], dst_parts[:2], n_mesh),
        _sc_scatter(ef_parts[2:], dst_parts[2:], n_mesh),
    ]
    mesh_out = _node_mlp(mesh_node_features, partials, nW1[:D], nW1[D:],
                         nb1, nW2, nb2, ng, nbn)
    grid_out = _resid_mlp(grid_node_features, gW1, gb1, gW2, gb2, gg, gbn,
                          blk=2000)
    return (grid_out, mesh_out)
